# baseline (device time: 19537 ns/iter reference)
import jax
import jax.numpy as jnp
from jax import lax
from jax.experimental import pallas as pl
from jax.experimental.pallas import tpu as pltpu

import os

N_DEV = 8
BLOCK_M = int(os.environ.get("KERNEL_BLOCK_M", "2048"))
_NO_RDMA_PROBE = os.environ.get("KERNEL_NO_RDMA") == "1"
_NO_COMBINE_PROBE = os.environ.get("KERNEL_NO_COMBINE") == "1"


def kernel(x):
    m_per, n = x.shape
    n_blocks = m_per // BLOCK_M

    def body(x_ref, out_ref, gat_ref, send_sems, recv_sems):
        my = lax.axis_index("i")
        b = pl.program_id(0)
        barrier_sem = pltpu.get_barrier_semaphore()

        @pl.when(b == 0)
        def _():
            for o in range(1, N_DEV):
                pl.semaphore_signal(
                    barrier_sem, inc=1,
                    device_id=(lax.rem(my + o, N_DEV),),
                    device_id_type=pl.DeviceIdType.MESH,
                )

        xb = x_ref[...]
        bv = jnp.max(xb, axis=0)
        iota = lax.broadcasted_iota(jnp.int32, (BLOCK_M, n), 0)
        bidx = jnp.min(
            jnp.where(xb == bv[None, :], iota, jnp.int32(BLOCK_M)), axis=0
        )
        gidx = (bidx + b * BLOCK_M + my * m_per).astype(jnp.float32)

        @pl.when(b == 0)
        def _():
            gat_ref[0, 0, :] = bv
            gat_ref[0, 1, :] = gidx

        @pl.when(b > 0)
        def _():
            rv = gat_ref[0, 0, :]
            better = bv > rv
            gat_ref[0, 0, :] = jnp.where(better, bv, rv)
            gat_ref[0, 1, :] = jnp.where(better, gidx, gat_ref[0, 1, :])

        @pl.when(b == max(n_blocks - 2, 0))
        def _():
            pl.semaphore_wait(barrier_sem, N_DEV - 1)

        @pl.when(b == n_blocks - 1)
        def _():
            if _NO_RDMA_PROBE:
                return

            sends = []
            for o in range(1, N_DEV):
                rdma = pltpu.make_async_remote_copy(
                    src_ref=gat_ref.at[0],
                    dst_ref=gat_ref.at[N_DEV - o],
                    send_sem=send_sems.at[o - 1],
                    recv_sem=recv_sems.at[o - 1],
                    device_id=(lax.rem(my + o, N_DEV),),
                    device_id_type=pl.DeviceIdType.MESH,
                )
                rdma.start()
                sends.append(rdma)

            for d in range(1, N_DEV):
                recv = pltpu.make_async_remote_copy(
                    src_ref=gat_ref.at[0],
                    dst_ref=gat_ref.at[d],
                    send_sem=send_sems.at[0],
                    recv_sem=recv_sems.at[N_DEV - 1 - d],
                    device_id=(my,),
                    device_id_type=pl.DeviceIdType.MESH,
                )
                recv.wait_recv()
            if _NO_COMBINE_PROBE:
                for rdma in sends:
                    rdma.wait_send()
                return

            allv = gat_ref[:, 0, :]
            alli = gat_ref[:, 1, :]
            best = jnp.max(allv, axis=0)
            out_ref[0, :] = best
            out_ref[1, :] = jnp.min(
                jnp.where(allv == best[None, :], alli, jnp.float32(4e9)),
                axis=0,
            )
            for rdma in sends:
                rdma.wait_send()

    return pl.pallas_call(
        body,
        grid=(n_blocks,),
        out_shape=jax.ShapeDtypeStruct((2, n), jnp.float32),
        in_specs=[
            pl.BlockSpec((BLOCK_M, n), lambda i: (i, 0),
                         memory_space=pltpu.VMEM)
        ],
        out_specs=pl.BlockSpec((2, n), lambda i: (0, 0),
                               memory_space=pltpu.VMEM),
        scratch_shapes=[
            pltpu.VMEM((N_DEV, 2, n), jnp.float32),
            pltpu.SemaphoreType.DMA((N_DEV - 1,)),
            pltpu.SemaphoreType.DMA((N_DEV - 1,)),
        ],
        compiler_params=pltpu.CompilerParams(collective_id=0),
    )(x)
